# Initial kernel scaffold; baseline (speedup 1.0000x reference)
import jax
import jax.numpy as jnp
from jax.experimental import pallas as pl


def kernel(items, items_emb_grad, client_losses, items_emb_weight):
    raise NotImplementedError("write your pallas kernel here")


# speedup vs baseline: 74.1085x; 74.1085x over previous
import jax
import jax.numpy as jnp
from jax import lax
from jax.experimental import pallas as pl
from jax.experimental.pallas import tpu as pltpu
from jax.experimental.pallas import tpu_sc as plsc

M_ITEM = 100000
DIM = 32
N_CLIENTS = 26
N_ITEMS = 200
LR = 0.01

NW = 32
ROWS_PER_W = M_ITEM // NW
WORDS_PER_W = ROWS_PER_W * DIM
N_ITEMS_PAD = 208
N_GROUPS = N_ITEMS_PAD // 16
GRAD_WORDS = N_ITEMS_PAD * DIM
C_PAD = 32


def _body(items_hbm, grad_hbm, losses_hbm, weight_hbm, out_hbm,
          losses_v, items_v, grad_v, slab_v, slab_sem):
    wid = lax.axis_index("s") * 2 + lax.axis_index("c")
    wbase = wid * WORDS_PER_W
    rbase = wid * ROWS_PER_W

    slab_cp = pltpu.make_async_copy(
        weight_hbm.at[pl.ds(wbase, WORDS_PER_W)], slab_v, slab_sem)
    slab_cp.start()

    pltpu.sync_copy(losses_hbm, losses_v)
    lane = lax.iota(jnp.int32, 16)
    v0 = losses_v[pl.ds(0, 16)]
    v1 = losses_v[pl.ds(16, 16)]
    valid1 = lane < (N_CLIENTS - 16)
    half = (jnp.sum(jnp.where(valid1, v1, 0.0)) + jnp.sum(v0)) * 0.5

    c_star = jnp.int32(0)
    for i in range(N_CLIENTS):
        src = v0 if i < 16 else v1
        li = jnp.sum(jnp.where(lane == (i % 16), src, 0.0))
        p0 = (v0 < li) | ((v0 == li) & (lane < i))
        p1 = (v1 < li) | ((v1 == li) & ((lane + 16) < i))
        s = jnp.sum(jnp.where(p0, v0, 0.0)) + jnp.sum(jnp.where(p1, v1, 0.0))
        sel = (s < half) & (s + li >= half)
        c_star = jnp.where(sel, jnp.int32(i), c_star)

    pltpu.sync_copy(items_hbm.at[pl.ds(c_star * N_ITEMS_PAD, N_ITEMS_PAD)],
                    items_v)
    pltpu.sync_copy(grad_hbm.at[pl.ds(c_star * (N_ITEMS * DIM), N_ITEMS * DIM)],
                    grad_v.at[pl.ds(0, N_ITEMS * DIM)])

    slab_cp.wait()

    for g in range(N_GROUPS):
        iv = items_v[pl.ds(g * 16, 16)]
        mask = (iv >= rbase) & (iv < rbase + ROWS_PER_W)
        cnt = jnp.max(plsc.all_reduce_population_count(mask))

        @pl.when(cnt > 0)
        def _patch(iv=iv, mask=mask, g=g):
            fbase = jnp.where(mask, (iv - rbase) * DIM, 0)
            kflat = (lane + g * 16) * DIM
            for c in range(DIM):
                gval = plsc.load_gather(grad_v, [kflat + c], mask=mask)
                cur = plsc.load_gather(slab_v, [fbase + c], mask=mask)
                plsc.store_scatter(slab_v, [fbase + c], cur - LR * gval,
                                   mask=mask)

    pltpu.sync_copy(slab_v, out_hbm.at[pl.ds(wbase, WORDS_PER_W)])


@jax.jit
def _run(items_p, grads, losses_p, weight):
    mesh = plsc.VectorSubcoreMesh(core_axis_name="c", subcore_axis_name="s",
                                  num_cores=2, num_subcores=16)
    return pl.kernel(
        _body,
        out_type=jax.ShapeDtypeStruct((M_ITEM * DIM,), jnp.float32),
        mesh=mesh,
        compiler_params=pltpu.CompilerParams(needs_layout_passes=False),
        scratch_types=[
            pltpu.VMEM((C_PAD,), jnp.float32),
            pltpu.VMEM((N_ITEMS_PAD,), jnp.int32),
            pltpu.VMEM((GRAD_WORDS,), jnp.float32),
            pltpu.VMEM((WORDS_PER_W,), jnp.float32),
            pltpu.SemaphoreType.DMA,
        ],
    )(items_p, grads, losses_p, weight)


def kernel(items, items_emb_grad, client_losses, items_emb_weight):
    items_p = jnp.pad(items, ((0, 0), (0, N_ITEMS_PAD - N_ITEMS)),
                      constant_values=-1).reshape(-1)
    losses_p = jnp.pad(client_losses, (0, C_PAD - N_CLIENTS),
                       constant_values=jnp.inf)
    grads_f = items_emb_grad.reshape(-1)
    weight_f = items_emb_weight.reshape(-1)
    out = _run(items_p, grads_f, losses_p, weight_f)
    return out.reshape(M_ITEM, DIM)


# speedup vs baseline: 82.1657x; 1.1087x over previous
import jax
import jax.numpy as jnp
from jax import lax
from jax.experimental import pallas as pl
from jax.experimental.pallas import tpu as pltpu
from jax.experimental.pallas import tpu_sc as plsc

M_ITEM = 100000
DIM = 32
N_CLIENTS = 26
N_ITEMS = 200
LR = 0.01

NW = 32
ROWS_MAIN = 3120
CHUNK = 312
N_CHUNKS = ROWS_MAIN // CHUNK
TAIL_BASE = ROWS_MAIN * NW
TAIL_ROWS = 8
N_TAIL_W = (M_ITEM - TAIL_BASE) // TAIL_ROWS
N_ITEMS_PAD = 208
N_GROUPS = N_ITEMS_PAD // 16
C_PAD = 32


def _patch_chunk(items_v, grad_v, buf, lane, lo, hi):
    def group_body(g, carry):
        iv = items_v[pl.ds(g * 16, 16)]
        in_c = (iv >= lo) & (iv < hi)
        cnt = jnp.max(plsc.all_reduce_population_count(in_c))

        @pl.when(cnt > 0)
        def _():
            rows = jnp.where(in_c, iv - lo, 0)
            kvec = lane + g * 16
            for c in range(DIM):
                colv = jnp.full((16,), c, jnp.int32)
                gval = plsc.load_gather(grad_v, [kvec, colv], mask=in_c)
                cur = plsc.load_gather(buf, [rows, colv], mask=in_c)
                plsc.store_scatter(buf, [rows, colv], cur - LR * gval,
                                   mask=in_c)
        return carry

    hit = jnp.int32(0)
    for g in range(N_GROUPS):
        iv = items_v[pl.ds(g * 16, 16)]
        in_c = (iv >= lo) & (iv < hi)
        hit = hit | jnp.max(plsc.all_reduce_population_count(in_c))

    @pl.when(hit > 0)
    def _():
        lax.fori_loop(0, N_GROUPS, group_body, jnp.int32(0))


def _body(items_hbm, grad_hbm, losses_hbm, weight_hbm, out_hbm,
          losses_v, items_v, grad_v, buf0, buf1, tbuf,
          in_sem0, in_sem1, out_sem0, out_sem1, tail_sem):
    wid = lax.axis_index("s") * 2 + lax.axis_index("c")
    base_a = wid * ROWS_MAIN
    base_b = TAIL_BASE + wid * TAIL_ROWS
    has_tail = wid < N_TAIL_W
    bufs = (buf0, buf1)
    in_sems = (in_sem0, in_sem1)
    out_sems = (out_sem0, out_sem1)

    def in_cp(ci):
        return pltpu.make_async_copy(
            weight_hbm.at[pl.ds(base_a + ci * CHUNK, CHUNK)],
            bufs[ci % 2], in_sems[ci % 2])

    def out_cp(ci):
        return pltpu.make_async_copy(
            bufs[ci % 2], out_hbm.at[pl.ds(base_a + ci * CHUNK, CHUNK)],
            out_sems[ci % 2])

    in_cp(0).start()
    in_cp(1).start()

    tail_in = pltpu.make_async_copy(
        weight_hbm.at[pl.ds(base_b, TAIL_ROWS)], tbuf, tail_sem)

    @pl.when(has_tail)
    def _tail_start():
        tail_in.start()

    pltpu.sync_copy(losses_hbm, losses_v)
    lane = lax.iota(jnp.int32, 16)
    v0 = losses_v[pl.ds(0, 16)]
    v1 = losses_v[pl.ds(16, 16)]
    valid1 = lane < (N_CLIENTS - 16)
    half = (jnp.sum(jnp.where(valid1, v1, 0.0)) + jnp.sum(v0)) * 0.5

    c_star = jnp.int32(0)
    for i in range(N_CLIENTS):
        src = v0 if i < 16 else v1
        li = jnp.sum(jnp.where(lane == (i % 16), src, 0.0))
        p0 = (v0 < li) | ((v0 == li) & (lane < i))
        p1 = (v1 < li) | ((v1 == li) & ((lane + 16) < i))
        s = jnp.sum(jnp.where(p0, v0, 0.0)) + jnp.sum(jnp.where(p1, v1, 0.0))
        sel = (s < half) & (s + li >= half)
        c_star = jnp.where(sel, jnp.int32(i), c_star)

    pltpu.sync_copy(items_hbm.at[pl.ds(c_star * N_ITEMS_PAD, N_ITEMS_PAD)],
                    items_v)
    pltpu.sync_copy(grad_hbm.at[c_star], grad_v.at[pl.ds(0, N_ITEMS)])

    for ci in range(N_CHUNKS):
        in_cp(ci).wait()
        _patch_chunk(items_v, grad_v, bufs[ci % 2], lane,
                     base_a + ci * CHUNK, base_a + (ci + 1) * CHUNK)
        out_cp(ci).start()
        if ci + 2 < N_CHUNKS:
            out_cp(ci).wait()
            in_cp(ci + 2).start()

    @pl.when(has_tail)
    def _tail_done():
        tail_in.wait()
        _patch_chunk(items_v, grad_v, tbuf, lane, base_b, base_b + TAIL_ROWS)
        pltpu.sync_copy(tbuf, out_hbm.at[pl.ds(base_b, TAIL_ROWS)])

    out_cp(N_CHUNKS - 2).wait()
    out_cp(N_CHUNKS - 1).wait()


@jax.jit
def _run(items_p, grads, losses_p, weight):
    mesh = plsc.VectorSubcoreMesh(core_axis_name="c", subcore_axis_name="s",
                                  num_cores=2, num_subcores=16)
    return pl.kernel(
        _body,
        out_type=jax.ShapeDtypeStruct((M_ITEM, DIM), jnp.float32),
        mesh=mesh,
        compiler_params=pltpu.CompilerParams(needs_layout_passes=False),
        scratch_types=[
            pltpu.VMEM((C_PAD,), jnp.float32),
            pltpu.VMEM((N_ITEMS_PAD,), jnp.int32),
            pltpu.VMEM((N_ITEMS_PAD, DIM), jnp.float32),
            pltpu.VMEM((CHUNK, DIM), jnp.float32),
            pltpu.VMEM((CHUNK, DIM), jnp.float32),
            pltpu.VMEM((TAIL_ROWS, DIM), jnp.float32),
            pltpu.SemaphoreType.DMA,
            pltpu.SemaphoreType.DMA,
            pltpu.SemaphoreType.DMA,
            pltpu.SemaphoreType.DMA,
            pltpu.SemaphoreType.DMA,
        ],
    )(items_p, grads, losses_p, weight)


def kernel(items, items_emb_grad, client_losses, items_emb_weight):
    items_p = jnp.pad(items, ((0, 0), (0, N_ITEMS_PAD - N_ITEMS)),
                      constant_values=-1).reshape(-1)
    losses_p = jnp.pad(client_losses, (0, C_PAD - N_CLIENTS),
                       constant_values=jnp.inf)
    return _run(items_p, items_emb_grad, losses_p, items_emb_weight)
